# baseline (speedup 1.0000x reference)
import functools

import jax
import jax.numpy as jnp
import numpy as np
from jax import lax
from jax.experimental import pallas as pl
from jax.experimental.pallas import tpu as pltpu
from jax.experimental.pallas import tpu_sc as plsc

_SPARSITY = 0.15
_GAIN = 3.0
_EPS = 1e-5

_B = 128
_N = 32768
_K = max(int(_N * _SPARSITY), 2)
_NVREG = _N // 16
_MIN32 = np.int32(-2147483648)
_M7F = np.int32(0x7FFFFFFF)

_GDN = lax.GatherDimensionNumbers(
    offset_dims=(), collapsed_slice_dims=(0,), start_index_map=(0,))


def _perm(v, idx):
    return lax.gather(v, idx[:, None], dimension_numbers=_GDN,
                      slice_sizes=(1,),
                      mode=lax.GatherScatterMode.PROMISE_IN_BOUNDS)


def _butterfly(v, lanes, op):
    for sh in (8, 4, 2, 1):
        v = op(v, _perm(v, lanes ^ sh))
    return v


def _suffix16(v, lanes, zero):
    for sh in (1, 2, 4, 8):
        shifted = _perm(v, jnp.minimum(lanes + sh, 15))
        v = v + jnp.where(lanes + sh > 15, zero, shifted)
    return v


def _sortable(bits):
    return jnp.where(bits >= 0, bits, bits ^ _M7F) ^ _MIN32


def _sc_body(x_hbm, g_hbm, b_hbm, o_hbm, sbuf, gbuf, bbuf, hist):
    wid = lax.axis_index("s") * 2 + lax.axis_index("c")

    pltpu.sync_copy(g_hbm, gbuf)
    pltpu.sync_copy(b_hbm, bbuf)
    lanes = lax.iota(jnp.int32, 16)
    izero = jnp.zeros((16,), jnp.int32)
    ones = jnp.ones((16,), jnp.int32)

    def do_row(row, _):
        pltpu.sync_copy(x_hbm.at[row], sbuf)

        def stats(j, carry):
            accs = list(carry)
            for u in range(8):
                v = sbuf[pl.ds(j * 128 + u * 16, 16)]
                accs[u] = accs[u] + v
                accs[8 + u] = accs[8 + u] + v * v
            return tuple(accs)

        z16 = jnp.zeros((16,), jnp.float32)
        accs = lax.fori_loop(0, _NVREG // 8, stats, (z16,) * 16)
        acc = accs[0]
        acc2 = accs[8]
        for u in range(1, 8):
            acc = acc + accs[u]
            acc2 = acc2 + accs[8 + u]
        mean = _butterfly(acc, lanes, jnp.add) * (1.0 / _N)
        ssq = _butterfly(acc2, lanes, jnp.add)
        var = jnp.maximum(ssq * (1.0 / _N) - mean * mean, 0.0) + _EPS
        vb = lax.bitcast_convert_type(var, jnp.int32)
        y = lax.bitcast_convert_type(np.int32(0x5F3759DF) - (vb >> 1),
                                     jnp.float32)
        for _i in range(4):
            y = y * (1.5 - 0.5 * var * y * y)
        istd = y

        def zero_hist(j, _c):
            for u in range(8):
                hist[pl.ds(j * 128 + u * 16, 16)] = izero
            return 0

        lanes16 = lanes * 16
        prefix = izero
        rank = jnp.full((16,), _K, jnp.int32)
        m1 = jnp.full((16,), _MIN32, jnp.int32)
        m2 = jnp.full((16,), _MIN32, jnp.int32)

        for level in range(4):
            sh = 24 - 8 * level
            lax.fori_loop(0, 34, zero_hist, 0)

            if level == 0:
                def hpass0(j, carry):
                    cm1, cm2 = carry
                    for u in range(8):
                        sl = pl.ds(j * 128 + u * 16, 16)
                        xn = (sbuf[sl] - mean) * istd * gbuf[sl] + bbuf[sl]
                        bits = lax.bitcast_convert_type(xn, jnp.int32)
                        s = _sortable(bits)
                        sbuf[sl] = lax.bitcast_convert_type(s, jnp.float32)
                        b = lax.shift_right_logical(s, 24)
                        plsc.addupdate_scatter(hist, [(b << 4) | lanes], ones)
                        plsc.addupdate_scatter(
                            hist, [4096 + ((b & 0xF0) | lanes)], ones)
                        sb = s ^ _MIN32
                        cm2 = jnp.maximum(cm2, jnp.minimum(cm1, sb))
                        cm1 = jnp.maximum(cm1, sb)
                    return cm1, cm2

                m1, m2 = lax.fori_loop(0, _NVREG // 8, hpass0, (m1, m2))
            else:
                pref = prefix

                def hpass(j, _c):
                    for u in range(8):
                        s = lax.bitcast_convert_type(
                            sbuf[pl.ds(j * 128 + u * 16, 16)], jnp.int32)
                        b = lax.shift_right_logical(s, sh) & 255
                        keep = lax.shift_right_logical(s, sh + 8) == pref
                        plsc.addupdate_scatter(hist, [(b << 4) | lanes],
                                               ones, mask=keep)
                        plsc.addupdate_scatter(
                            hist, [4096 + ((b & 0xF0) | lanes)], ones,
                            mask=keep)
                    return 0

                lax.fori_loop(0, _NVREG // 8, hpass, 0)

            G = izero
            for l in range(16):
                G = G + plsc.load_gather(hist, [4096 + lanes16 + l])
            S = _suffix16(G, lanes, izero)
            hitg = S >= rank
            grp = plsc.all_reduce_population_count(hitg) - 1
            above_g = _butterfly(jnp.where(hitg, 0, S), lanes, jnp.maximum)

            bt = izero
            base = grp * 256 + lanes16
            for l in range(16):
                bt = bt + plsc.load_gather(hist, [base + l])
            rank2 = rank - above_g
            S2 = _suffix16(bt, lanes, izero)
            hitb = S2 >= rank2
            bloc = plsc.all_reduce_population_count(hitb) - 1
            above_b = _butterfly(jnp.where(hitb, 0, S2), lanes, jnp.maximum)
            prefix = (prefix << 8) | ((grp << 4) | bloc)
            rank = rank2 - above_b

        thr_b = prefix ^ _MIN32

        m1s = _butterfly(m1, lanes, jnp.maximum)
        c1 = plsc.all_reduce_population_count(m1 == m1s)
        mbig = jnp.full((16,), np.int32(-2147483647), jnp.int32)
        strict2 = _butterfly(jnp.where(m1 == m1s, mbig, m1), lanes,
                             jnp.maximum)
        m2s = _butterfly(m2, lanes, jnp.maximum)
        second = jnp.where(c1 >= 2, m1s, jnp.maximum(strict2, m2s))
        u1 = jnp.where(m1s >= 0, m1s, m1s ^ _M7F)
        u2 = jnp.where(second >= 0, second, second ^ _M7F)
        f1 = lax.bitcast_convert_type(u1, jnp.float32)
        f2 = lax.bitcast_convert_type(u2, jnp.float32)
        gain = _GAIN / (1.0 + jnp.exp(f2 - f1)) + 1.0

        def opass(j, _c):
            for u in range(8):
                sl = pl.ds(j * 128 + u * 16, 16)
                s = lax.bitcast_convert_type(sbuf[sl], jnp.int32)
                sb = s ^ _MIN32
                keep = sb >= thr_b
                w = jnp.where(sb >= 0, sb, sb ^ _M7F)
                xn = lax.bitcast_convert_type(w, jnp.float32)
                sbuf[sl] = jnp.where(keep, xn * gain, 0.0)
            return 0

        lax.fori_loop(0, _NVREG // 8, opass, 0)
        pltpu.sync_copy(sbuf, o_hbm.at[row])
        return 0

    lax.fori_loop(wid * 4, wid * 4 + 4, do_row, 0)


@jax.jit
def kernel(x_input, ln_gamma, ln_beta):
    mesh = plsc.VectorSubcoreMesh(core_axis_name="c", subcore_axis_name="s")
    fn = pl.kernel(
        _sc_body,
        out_type=jax.ShapeDtypeStruct((_B, _N), jnp.float32),
        mesh=mesh,
        compiler_params=pltpu.CompilerParams(needs_layout_passes=False),
        scratch_types=[
            pltpu.VMEM((_N,), jnp.float32),
            pltpu.VMEM((_N,), jnp.float32),
            pltpu.VMEM((_N,), jnp.float32),
            pltpu.VMEM((4352,), jnp.int32),
        ],
    )
    return fn(x_input, ln_gamma, ln_beta)


# speedup vs baseline: 1.0572x; 1.0572x over previous
import functools

import jax
import jax.numpy as jnp
import numpy as np
from jax import lax
from jax.experimental import pallas as pl
from jax.experimental.pallas import tpu as pltpu
from jax.experimental.pallas import tpu_sc as plsc

_SPARSITY = 0.15
_GAIN = 3.0
_EPS = 1e-5

_B = 128
_N = 32768
_K = max(int(_N * _SPARSITY), 2)
_NVREG = _N // 16
_MIN32 = np.int32(-2147483648)
_M7F = np.int32(0x7FFFFFFF)

_GDN = lax.GatherDimensionNumbers(
    offset_dims=(), collapsed_slice_dims=(0,), start_index_map=(0,))


def _perm(v, idx):
    return lax.gather(v, idx[:, None], dimension_numbers=_GDN,
                      slice_sizes=(1,),
                      mode=lax.GatherScatterMode.PROMISE_IN_BOUNDS)


def _butterfly(v, lanes, op):
    for sh in (8, 4, 2, 1):
        v = op(v, _perm(v, lanes ^ sh))
    return v


def _suffix16(v, lanes, zero):
    for sh in (1, 2, 4, 8):
        shifted = _perm(v, jnp.minimum(lanes + sh, 15))
        v = v + jnp.where(lanes + sh > 15, zero, shifted)
    return v


def _sortable(bits):
    return jnp.where(bits >= 0, bits, bits ^ _M7F) ^ _MIN32


def _sc_body(x_hbm, g_hbm, b_hbm, o_hbm, sbuf, gbuf, bbuf, hist):
    wid = lax.axis_index("s") * 2 + lax.axis_index("c")

    pltpu.sync_copy(g_hbm, gbuf)
    pltpu.sync_copy(b_hbm, bbuf)
    lanes = lax.iota(jnp.int32, 16)
    izero = jnp.zeros((16,), jnp.int32)
    ones = jnp.ones((16,), jnp.int32)

    def do_row(row, _):
        pltpu.sync_copy(x_hbm.at[row], sbuf)

        def stats(j, carry):
            accs = list(carry)
            for u in range(8):
                v = sbuf[pl.ds(j * 128 + u * 16, 16)]
                accs[u] = accs[u] + v
                accs[8 + u] = accs[8 + u] + v * v
            return tuple(accs)

        z16 = jnp.zeros((16,), jnp.float32)
        accs = lax.fori_loop(0, _NVREG // 8, stats, (z16,) * 16)
        acc = accs[0]
        acc2 = accs[8]
        for u in range(1, 8):
            acc = acc + accs[u]
            acc2 = acc2 + accs[8 + u]
        mean = _butterfly(acc, lanes, jnp.add) * (1.0 / _N)
        ssq = _butterfly(acc2, lanes, jnp.add)
        var = jnp.maximum(ssq * (1.0 / _N) - mean * mean, 0.0) + _EPS
        vb = lax.bitcast_convert_type(var, jnp.int32)
        y = lax.bitcast_convert_type(np.int32(0x5F3759DF) - (vb >> 1),
                                     jnp.float32)
        for _i in range(4):
            y = y * (1.5 - 0.5 * var * y * y)
        istd = y

        def zero_hist(j, _c):
            for u in range(8):
                hist[pl.ds(j * 128 + u * 16, 16)] = izero
            return 0

        lanes16 = lanes * 16
        prefix = izero
        rank = jnp.full((16,), _K, jnp.int32)
        m1 = jnp.full((16,), _MIN32, jnp.int32)
        m2 = jnp.full((16,), _MIN32, jnp.int32)

        for level in range(4):
            sh = 24 - 8 * level
            lax.fori_loop(0, 34, zero_hist, 0)

            if level == 0:
                def hpass0(j, carry):
                    cm1, cm2 = carry
                    for u in range(8):
                        sl = pl.ds(j * 128 + u * 16, 16)
                        xn = (sbuf[sl] - mean) * istd * gbuf[sl] + bbuf[sl]
                        bits = lax.bitcast_convert_type(xn, jnp.int32)
                        s = _sortable(bits)
                        sbuf[sl] = lax.bitcast_convert_type(s, jnp.float32)
                        b = lax.shift_right_logical(s, 24)
                        plsc.addupdate_scatter(hist, [(b << 4) | lanes], ones)
                        sb = s ^ _MIN32
                        cm2 = jnp.maximum(cm2, jnp.minimum(cm1, sb))
                        cm1 = jnp.maximum(cm1, sb)
                    return cm1, cm2

                m1, m2 = lax.fori_loop(0, _NVREG // 8, hpass0, (m1, m2))
            else:
                pref = prefix

                def hpass(j, _c):
                    for u in range(8):
                        s = lax.bitcast_convert_type(
                            sbuf[pl.ds(j * 128 + u * 16, 16)], jnp.int32)
                        b = lax.shift_right_logical(s, sh) & 255
                        keep = lax.shift_right_logical(s, sh + 8) == pref
                        plsc.addupdate_scatter(hist, [(b << 4) | lanes],
                                               ones, mask=keep)
                    return 0

                lax.fori_loop(0, _NVREG // 8, hpass, 0)

            for g in range(16):
                cv = hist[pl.ds(g * 256, 16)]
                for j in range(1, 16):
                    cv = cv + hist[pl.ds(g * 256 + j * 16, 16)]
                hist[pl.ds(4096 + g * 16, 16)] = cv
            G = izero
            for l in range(16):
                G = G + plsc.load_gather(hist, [4096 + lanes16 + l])
            S = _suffix16(G, lanes, izero)
            hitg = S >= rank
            grp = plsc.all_reduce_population_count(hitg) - 1
            above_g = _butterfly(jnp.where(hitg, 0, S), lanes, jnp.maximum)

            bt = izero
            base = grp * 256 + lanes16
            for l in range(16):
                bt = bt + plsc.load_gather(hist, [base + l])
            rank2 = rank - above_g
            S2 = _suffix16(bt, lanes, izero)
            hitb = S2 >= rank2
            bloc = plsc.all_reduce_population_count(hitb) - 1
            above_b = _butterfly(jnp.where(hitb, 0, S2), lanes, jnp.maximum)
            prefix = (prefix << 8) | ((grp << 4) | bloc)
            rank = rank2 - above_b

        thr_b = prefix ^ _MIN32

        m1s = _butterfly(m1, lanes, jnp.maximum)
        c1 = plsc.all_reduce_population_count(m1 == m1s)
        mbig = jnp.full((16,), np.int32(-2147483647), jnp.int32)
        strict2 = _butterfly(jnp.where(m1 == m1s, mbig, m1), lanes,
                             jnp.maximum)
        m2s = _butterfly(m2, lanes, jnp.maximum)
        second = jnp.where(c1 >= 2, m1s, jnp.maximum(strict2, m2s))
        u1 = jnp.where(m1s >= 0, m1s, m1s ^ _M7F)
        u2 = jnp.where(second >= 0, second, second ^ _M7F)
        f1 = lax.bitcast_convert_type(u1, jnp.float32)
        f2 = lax.bitcast_convert_type(u2, jnp.float32)
        gain = _GAIN / (1.0 + jnp.exp(f2 - f1)) + 1.0

        def opass(j, _c):
            for u in range(8):
                sl = pl.ds(j * 128 + u * 16, 16)
                s = lax.bitcast_convert_type(sbuf[sl], jnp.int32)
                sb = s ^ _MIN32
                keep = sb >= thr_b
                w = jnp.where(sb >= 0, sb, sb ^ _M7F)
                xn = lax.bitcast_convert_type(w, jnp.float32)
                sbuf[sl] = jnp.where(keep, xn * gain, 0.0)
            return 0

        lax.fori_loop(0, _NVREG // 8, opass, 0)
        pltpu.sync_copy(sbuf, o_hbm.at[row])
        return 0

    lax.fori_loop(wid * 4, wid * 4 + 4, do_row, 0)


@jax.jit
def kernel(x_input, ln_gamma, ln_beta):
    mesh = plsc.VectorSubcoreMesh(core_axis_name="c", subcore_axis_name="s")
    fn = pl.kernel(
        _sc_body,
        out_type=jax.ShapeDtypeStruct((_B, _N), jnp.float32),
        mesh=mesh,
        compiler_params=pltpu.CompilerParams(needs_layout_passes=False),
        scratch_types=[
            pltpu.VMEM((_N,), jnp.float32),
            pltpu.VMEM((_N,), jnp.float32),
            pltpu.VMEM((_N,), jnp.float32),
            pltpu.VMEM((4352,), jnp.int32),
        ],
    )
    return fn(x_input, ln_gamma, ln_beta)


# speedup vs baseline: 2.8461x; 2.6920x over previous
import functools

import jax
import jax.numpy as jnp
import numpy as np
from jax import lax
from jax.experimental import pallas as pl
from jax.experimental.pallas import tpu as pltpu
from jax.experimental.pallas import tpu_sc as plsc

_SPARSITY = 0.15
_GAIN = 3.0
_EPS = 1e-5

_B = 32
_N = 32768
_K = max(int(_N * _SPARSITY), 2)
_NVREG = _N // 16
_MIN32 = np.int32(-2147483648)
_M7F = np.int32(0x7FFFFFFF)

_GDN = lax.GatherDimensionNumbers(
    offset_dims=(), collapsed_slice_dims=(0,), start_index_map=(0,))


def _perm(v, idx):
    return lax.gather(v, idx[:, None], dimension_numbers=_GDN,
                      slice_sizes=(1,),
                      mode=lax.GatherScatterMode.PROMISE_IN_BOUNDS)


def _butterfly(v, lanes, op):
    for sh in (8, 4, 2, 1):
        v = op(v, _perm(v, lanes ^ sh))
    return v


def _suffix16(v, lanes, zero):
    for sh in (1, 2, 4, 8):
        shifted = _perm(v, jnp.minimum(lanes + sh, 15))
        v = v + jnp.where(lanes + sh > 15, zero, shifted)
    return v


def _sortable(bits):
    return jnp.where(bits >= 0, bits, bits ^ _M7F) ^ _MIN32


def _sc_body(x_hbm, g_hbm, b_hbm, o_hbm, sbuf, gbuf, bbuf, hist):
    wid = lax.axis_index("s") * 2 + lax.axis_index("c")

    pltpu.sync_copy(g_hbm, gbuf)
    pltpu.sync_copy(b_hbm, bbuf)
    lanes = lax.iota(jnp.int32, 16)
    izero = jnp.zeros((16,), jnp.int32)
    ones = jnp.ones((16,), jnp.int32)

    def do_row(row, _):
        pltpu.sync_copy(x_hbm.at[row], sbuf)

        def stats(j, carry):
            accs = list(carry)
            for u in range(8):
                v = sbuf[pl.ds(j * 128 + u * 16, 16)]
                accs[u] = accs[u] + v
                accs[8 + u] = accs[8 + u] + v * v
            return tuple(accs)

        z16 = jnp.zeros((16,), jnp.float32)
        accs = lax.fori_loop(0, _NVREG // 8, stats, (z16,) * 16)
        acc = accs[0]
        acc2 = accs[8]
        for u in range(1, 8):
            acc = acc + accs[u]
            acc2 = acc2 + accs[8 + u]
        mean = _butterfly(acc, lanes, jnp.add) * (1.0 / _N)
        ssq = _butterfly(acc2, lanes, jnp.add)
        var = jnp.maximum(ssq * (1.0 / _N) - mean * mean, 0.0) + _EPS
        vb = lax.bitcast_convert_type(var, jnp.int32)
        y = lax.bitcast_convert_type(np.int32(0x5F3759DF) - (vb >> 1),
                                     jnp.float32)
        for _i in range(4):
            y = y * (1.5 - 0.5 * var * y * y)
        istd = y

        def zero_hist(j, _c):
            for u in range(8):
                hist[pl.ds(j * 128 + u * 16, 16)] = izero
            return 0

        lanes16 = lanes * 16
        prefix = izero
        rank = jnp.full((16,), _K, jnp.int32)
        m1 = jnp.full((16,), _MIN32, jnp.int32)
        m2 = jnp.full((16,), _MIN32, jnp.int32)

        for level in range(4):
            sh = 24 - 8 * level
            lax.fori_loop(0, 34, zero_hist, 0)

            if level == 0:
                def hpass0(j, carry):
                    cm1, cm2 = carry
                    for u in range(8):
                        sl = pl.ds(j * 128 + u * 16, 16)
                        xn = (sbuf[sl] - mean) * istd * gbuf[sl] + bbuf[sl]
                        bits = lax.bitcast_convert_type(xn, jnp.int32)
                        s = _sortable(bits)
                        sbuf[sl] = lax.bitcast_convert_type(s, jnp.float32)
                        b = lax.shift_right_logical(s, 24)
                        plsc.addupdate_scatter(hist, [(b << 4) | lanes], ones)
                        sb = s ^ _MIN32
                        cm2 = jnp.maximum(cm2, jnp.minimum(cm1, sb))
                        cm1 = jnp.maximum(cm1, sb)
                    return cm1, cm2

                m1, m2 = lax.fori_loop(0, _NVREG // 8, hpass0, (m1, m2))
            else:
                pref = prefix

                def hpass(j, _c):
                    for u in range(8):
                        s = lax.bitcast_convert_type(
                            sbuf[pl.ds(j * 128 + u * 16, 16)], jnp.int32)
                        b = lax.shift_right_logical(s, sh) & 255
                        keep = lax.shift_right_logical(s, sh + 8) == pref
                        plsc.addupdate_scatter(hist, [(b << 4) | lanes],
                                               ones, mask=keep)
                    return 0

                lax.fori_loop(0, _NVREG // 8, hpass, 0)

            for g in range(16):
                cv = hist[pl.ds(g * 256, 16)]
                for j in range(1, 16):
                    cv = cv + hist[pl.ds(g * 256 + j * 16, 16)]
                hist[pl.ds(4096 + g * 16, 16)] = cv
            G = izero
            for l in range(16):
                G = G + plsc.load_gather(hist, [4096 + lanes16 + l])
            S = _suffix16(G, lanes, izero)
            hitg = S >= rank
            grp = plsc.all_reduce_population_count(hitg) - 1
            above_g = _butterfly(jnp.where(hitg, 0, S), lanes, jnp.maximum)

            bt = izero
            base = grp * 256 + lanes16
            for l in range(16):
                bt = bt + plsc.load_gather(hist, [base + l])
            rank2 = rank - above_g
            S2 = _suffix16(bt, lanes, izero)
            hitb = S2 >= rank2
            bloc = plsc.all_reduce_population_count(hitb) - 1
            above_b = _butterfly(jnp.where(hitb, 0, S2), lanes, jnp.maximum)
            prefix = (prefix << 8) | ((grp << 4) | bloc)
            rank = rank2 - above_b

        thr_b = prefix ^ _MIN32

        m1s = _butterfly(m1, lanes, jnp.maximum)
        c1 = plsc.all_reduce_population_count(m1 == m1s)
        mbig = jnp.full((16,), np.int32(-2147483647), jnp.int32)
        strict2 = _butterfly(jnp.where(m1 == m1s, mbig, m1), lanes,
                             jnp.maximum)
        m2s = _butterfly(m2, lanes, jnp.maximum)
        second = jnp.where(c1 >= 2, m1s, jnp.maximum(strict2, m2s))
        u1 = jnp.where(m1s >= 0, m1s, m1s ^ _M7F)
        u2 = jnp.where(second >= 0, second, second ^ _M7F)
        f1 = lax.bitcast_convert_type(u1, jnp.float32)
        f2 = lax.bitcast_convert_type(u2, jnp.float32)
        gain = _GAIN / (1.0 + jnp.exp(f2 - f1)) + 1.0

        def opass(j, _c):
            for u in range(8):
                sl = pl.ds(j * 128 + u * 16, 16)
                s = lax.bitcast_convert_type(sbuf[sl], jnp.int32)
                sb = s ^ _MIN32
                keep = sb >= thr_b
                w = jnp.where(sb >= 0, sb, sb ^ _M7F)
                xn = lax.bitcast_convert_type(w, jnp.float32)
                sbuf[sl] = jnp.where(keep, xn * gain, 0.0)
            return 0

        lax.fori_loop(0, _NVREG // 8, opass, 0)
        pltpu.sync_copy(sbuf, o_hbm.at[row])
        return 0

    do_row(wid, 0)


def _sc_kernel(x_input, ln_gamma, ln_beta):
    mesh = plsc.VectorSubcoreMesh(core_axis_name="c", subcore_axis_name="s")
    fn = pl.kernel(
        _sc_body,
        out_type=jax.ShapeDtypeStruct((_B, _N), jnp.float32),
        mesh=mesh,
        compiler_params=pltpu.CompilerParams(needs_layout_passes=False),
        scratch_types=[
            pltpu.VMEM((_N,), jnp.float32),
            pltpu.VMEM((_N,), jnp.float32),
            pltpu.VMEM((_N,), jnp.float32),
            pltpu.VMEM((4352,), jnp.int32),
        ],
    )
    return fn(x_input, ln_gamma, ln_beta)


def _tc_kernel(x_ref, g_ref, b_ref, o_ref, *, k):
    x = x_ref[...]
    rows = x.shape[0]

    mean = jnp.mean(x, axis=1, keepdims=True)
    xc = x - mean
    var = jnp.mean(xc * xc, axis=1, keepdims=True)
    xn = xc * jax.lax.rsqrt(var + _EPS)
    xn = xn * g_ref[...] + b_ref[...]

    u = jax.lax.bitcast_convert_type(xn, jnp.uint32)
    neg = u >= jnp.uint32(0x80000000)
    s = jnp.where(neg, ~u, u | jnp.uint32(0x80000000))

    lo0 = jnp.zeros((rows, 1), jnp.uint32)
    hi0 = jnp.full((rows, 1), 0xFFFFFFFF, jnp.uint32)

    def cond(carry):
        i, lo, hi = carry
        return jnp.logical_and(i < 32, jnp.logical_not(jnp.all(lo == hi)))

    def body(carry):
        i, lo, hi = carry
        d = hi - lo
        mid = lo + (d >> 1) + (d & jnp.uint32(1))
        cnt = jnp.sum((s >= mid).astype(jnp.int32), axis=1, keepdims=True)
        pred = cnt >= k
        lo = jnp.where(pred, mid, lo)
        hi = jnp.where(cnt == k, mid, jnp.where(pred, hi, mid - jnp.uint32(1)))
        return i + 1, lo, hi

    _, lo, _ = jax.lax.while_loop(cond, body, (jnp.int32(0), lo0, hi0))
    keep = s >= lo

    m1 = jnp.max(xn, axis=1, keepdims=True)
    is_max = xn == m1
    nmax = jnp.sum(is_max.astype(jnp.int32), axis=1, keepdims=True)
    m2_strict = jnp.max(jnp.where(is_max, -jnp.inf, xn), axis=1, keepdims=True)
    m2 = jnp.where(nmax >= 2, m1, m2_strict)
    gain = jax.nn.sigmoid(m1 - m2) * _GAIN + 1.0

    o_ref[...] = jnp.where(keep, xn * gain, 0.0)


def _tc_call(x_input, ln_gamma, ln_beta):
    b, n = x_input.shape
    k = max(int(n * _SPARSITY), 2)
    rb = 32
    grid = (b // rb,)
    body = functools.partial(_tc_kernel, k=k)
    return pl.pallas_call(
        body,
        grid=grid,
        in_specs=[
            pl.BlockSpec((rb, n), lambda i: (i, 0)),
            pl.BlockSpec((1, n), lambda i: (0, 0)),
            pl.BlockSpec((1, n), lambda i: (0, 0)),
        ],
        out_specs=pl.BlockSpec((rb, n), lambda i: (i, 0)),
        out_shape=jax.ShapeDtypeStruct((b, n), jnp.float32),
    )(x_input, ln_gamma.reshape(1, n), ln_beta.reshape(1, n))


@jax.jit
def kernel(x_input, ln_gamma, ln_beta):
    tc_out = _tc_call(x_input[:96], ln_gamma, ln_beta)
    sc_out = _sc_kernel(x_input[96:], ln_gamma, ln_beta)
    return jnp.concatenate([tc_out, sc_out], axis=0)


# speedup vs baseline: 2.8485x; 1.0009x over previous
import functools

import jax
import jax.numpy as jnp
import numpy as np
from jax import lax
from jax.experimental import pallas as pl
from jax.experimental.pallas import tpu as pltpu
from jax.experimental.pallas import tpu_sc as plsc

_SPARSITY = 0.15
_GAIN = 3.0
_EPS = 1e-5

_B = 32
_N = 32768
_K = max(int(_N * _SPARSITY), 2)
_NVREG = _N // 16
_MIN32 = np.int32(-2147483648)
_M7F = np.int32(0x7FFFFFFF)

_GDN = lax.GatherDimensionNumbers(
    offset_dims=(), collapsed_slice_dims=(0,), start_index_map=(0,))


def _perm(v, idx):
    return lax.gather(v, idx[:, None], dimension_numbers=_GDN,
                      slice_sizes=(1,),
                      mode=lax.GatherScatterMode.PROMISE_IN_BOUNDS)


def _butterfly(v, lanes, op):
    for sh in (8, 4, 2, 1):
        v = op(v, _perm(v, lanes ^ sh))
    return v


def _suffix16(v, lanes, zero):
    for sh in (1, 2, 4, 8):
        shifted = _perm(v, jnp.minimum(lanes + sh, 15))
        v = v + jnp.where(lanes + sh > 15, zero, shifted)
    return v


def _sortable(bits):
    return jnp.where(bits >= 0, bits, bits ^ _M7F) ^ _MIN32


def _sc_body(x_hbm, g_hbm, b_hbm, o_hbm, sbuf, gbuf, bbuf, hist):
    wid = lax.axis_index("s") * 2 + lax.axis_index("c")

    pltpu.sync_copy(g_hbm, gbuf)
    pltpu.sync_copy(b_hbm, bbuf)
    lanes = lax.iota(jnp.int32, 16)
    izero = jnp.zeros((16,), jnp.int32)
    ones = jnp.ones((16,), jnp.int32)

    def do_row(row, _):
        pltpu.sync_copy(x_hbm.at[row], sbuf)

        def stats(j, carry):
            accs = list(carry)
            for u in range(8):
                v = sbuf[pl.ds(j * 128 + u * 16, 16)]
                accs[u] = accs[u] + v
                accs[8 + u] = accs[8 + u] + v * v
            return tuple(accs)

        z16 = jnp.zeros((16,), jnp.float32)
        accs = lax.fori_loop(0, _NVREG // 8, stats, (z16,) * 16)
        acc = accs[0]
        acc2 = accs[8]
        for u in range(1, 8):
            acc = acc + accs[u]
            acc2 = acc2 + accs[8 + u]
        mean = _butterfly(acc, lanes, jnp.add) * (1.0 / _N)
        ssq = _butterfly(acc2, lanes, jnp.add)
        var = jnp.maximum(ssq * (1.0 / _N) - mean * mean, 0.0) + _EPS
        vb = lax.bitcast_convert_type(var, jnp.int32)
        y = lax.bitcast_convert_type(np.int32(0x5F3759DF) - (vb >> 1),
                                     jnp.float32)
        for _i in range(4):
            y = y * (1.5 - 0.5 * var * y * y)
        istd = y

        def zero_hist(j, _c):
            for u in range(8):
                hist[pl.ds(j * 128 + u * 16, 16)] = izero
            return 0

        lanes16 = lanes * 16
        prefix = izero
        rank = jnp.full((16,), _K, jnp.int32)
        m1 = jnp.full((16,), _MIN32, jnp.int32)
        m2 = jnp.full((16,), _MIN32, jnp.int32)

        for level in range(4):
            sh = 24 - 8 * level
            lax.fori_loop(0, 34, zero_hist, 0)

            if level == 0:
                def hpass0(j, carry):
                    cm1, cm2 = carry
                    for u in range(8):
                        sl = pl.ds(j * 128 + u * 16, 16)
                        xn = (sbuf[sl] - mean) * istd * gbuf[sl] + bbuf[sl]
                        bits = lax.bitcast_convert_type(xn, jnp.int32)
                        s = _sortable(bits)
                        sbuf[sl] = lax.bitcast_convert_type(s, jnp.float32)
                        b = lax.shift_right_logical(s, 24)
                        plsc.addupdate_scatter(hist, [(b << 4) | lanes], ones)
                        sb = s ^ _MIN32
                        cm2 = jnp.maximum(cm2, jnp.minimum(cm1, sb))
                        cm1 = jnp.maximum(cm1, sb)
                    return cm1, cm2

                m1, m2 = lax.fori_loop(0, _NVREG // 8, hpass0, (m1, m2))
            else:
                pref = prefix

                def hpass(j, _c):
                    for u in range(8):
                        s = lax.bitcast_convert_type(
                            sbuf[pl.ds(j * 128 + u * 16, 16)], jnp.int32)
                        b = lax.shift_right_logical(s, sh) & 255
                        keep = lax.shift_right_logical(s, sh + 8) == pref
                        plsc.addupdate_scatter(hist, [(b << 4) | lanes],
                                               ones, mask=keep)
                    return 0

                lax.fori_loop(0, _NVREG // 8, hpass, 0)

            for g in range(16):
                cv = hist[pl.ds(g * 256, 16)]
                for j in range(1, 16):
                    cv = cv + hist[pl.ds(g * 256 + j * 16, 16)]
                hist[pl.ds(4096 + g * 16, 16)] = cv
            G = izero
            for l in range(16):
                G = G + plsc.load_gather(hist, [4096 + lanes16 + l])
            S = _suffix16(G, lanes, izero)
            hitg = S >= rank
            grp = plsc.all_reduce_population_count(hitg) - 1
            above_g = _butterfly(jnp.where(hitg, 0, S), lanes, jnp.maximum)

            bt = izero
            base = grp * 256 + lanes16
            for l in range(16):
                bt = bt + plsc.load_gather(hist, [base + l])
            rank2 = rank - above_g
            S2 = _suffix16(bt, lanes, izero)
            hitb = S2 >= rank2
            bloc = plsc.all_reduce_population_count(hitb) - 1
            above_b = _butterfly(jnp.where(hitb, 0, S2), lanes, jnp.maximum)
            prefix = (prefix << 8) | ((grp << 4) | bloc)
            rank = rank2 - above_b

        thr_b = prefix ^ _MIN32

        m1s = _butterfly(m1, lanes, jnp.maximum)
        c1 = plsc.all_reduce_population_count(m1 == m1s)
        mbig = jnp.full((16,), np.int32(-2147483647), jnp.int32)
        strict2 = _butterfly(jnp.where(m1 == m1s, mbig, m1), lanes,
                             jnp.maximum)
        m2s = _butterfly(m2, lanes, jnp.maximum)
        second = jnp.where(c1 >= 2, m1s, jnp.maximum(strict2, m2s))
        u1 = jnp.where(m1s >= 0, m1s, m1s ^ _M7F)
        u2 = jnp.where(second >= 0, second, second ^ _M7F)
        f1 = lax.bitcast_convert_type(u1, jnp.float32)
        f2 = lax.bitcast_convert_type(u2, jnp.float32)
        gain = _GAIN / (1.0 + jnp.exp(f2 - f1)) + 1.0

        def opass(j, _c):
            for u in range(8):
                sl = pl.ds(j * 128 + u * 16, 16)
                s = lax.bitcast_convert_type(sbuf[sl], jnp.int32)
                sb = s ^ _MIN32
                keep = sb >= thr_b
                w = jnp.where(sb >= 0, sb, sb ^ _M7F)
                xn = lax.bitcast_convert_type(w, jnp.float32)
                sbuf[sl] = jnp.where(keep, xn * gain, 0.0)
            return 0

        lax.fori_loop(0, _NVREG // 8, opass, 0)
        pltpu.sync_copy(sbuf, o_hbm.at[row])
        return 0

    do_row(wid, 0)


def _sc_kernel(x_input, ln_gamma, ln_beta):
    mesh = plsc.VectorSubcoreMesh(core_axis_name="c", subcore_axis_name="s")
    fn = pl.kernel(
        _sc_body,
        out_type=jax.ShapeDtypeStruct((_B, _N), jnp.float32),
        mesh=mesh,
        compiler_params=pltpu.CompilerParams(needs_layout_passes=False),
        scratch_types=[
            pltpu.VMEM((_N,), jnp.float32),
            pltpu.VMEM((_N,), jnp.float32),
            pltpu.VMEM((_N,), jnp.float32),
            pltpu.VMEM((4352,), jnp.int32),
        ],
    )
    return fn(x_input, ln_gamma, ln_beta)


def _tc_kernel(x_ref, g_ref, b_ref, o_ref, *, k):
    x = x_ref[...]
    rows = x.shape[0]

    mean = jnp.mean(x, axis=1, keepdims=True)
    xc = x - mean
    var = jnp.mean(xc * xc, axis=1, keepdims=True)
    xn = xc * jax.lax.rsqrt(var + _EPS)
    xn = xn * g_ref[...] + b_ref[...]

    u = jax.lax.bitcast_convert_type(xn, jnp.uint32)
    neg = u >= jnp.uint32(0x80000000)
    s = jnp.where(neg, ~u, u | jnp.uint32(0x80000000))

    lo0 = jnp.zeros((rows, 1), jnp.uint32)
    hi0 = jnp.full((rows, 1), 0xFFFFFFFF, jnp.uint32)

    def cond(carry):
        i, lo, hi = carry
        return jnp.logical_and(i < 32, jnp.logical_not(jnp.all(lo == hi)))

    def body(carry):
        i, lo, hi = carry
        d = hi - lo
        mid = lo + (d >> 1) + (d & jnp.uint32(1))
        cnt = jnp.sum((s >= mid).astype(jnp.int32), axis=1, keepdims=True)
        pred = cnt >= k
        lo = jnp.where(pred, mid, lo)
        hi = jnp.where(cnt == k, mid, jnp.where(pred, hi, mid - jnp.uint32(1)))
        return i + 1, lo, hi

    _, lo, _ = jax.lax.while_loop(cond, body, (jnp.int32(0), lo0, hi0))
    keep = s >= lo

    m1 = jnp.max(xn, axis=1, keepdims=True)
    is_max = xn == m1
    nmax = jnp.sum(is_max.astype(jnp.int32), axis=1, keepdims=True)
    m2_strict = jnp.max(jnp.where(is_max, -jnp.inf, xn), axis=1, keepdims=True)
    m2 = jnp.where(nmax >= 2, m1, m2_strict)
    gain = jax.nn.sigmoid(m1 - m2) * _GAIN + 1.0

    o_ref[...] = jnp.where(keep, xn * gain, 0.0)


def _tc_call(x_input, ln_gamma, ln_beta):
    b, n = x_input.shape
    k = max(int(n * _SPARSITY), 2)
    rb = 32
    grid = (b // rb,)
    body = functools.partial(_tc_kernel, k=k)
    return pl.pallas_call(
        body,
        grid=grid,
        in_specs=[
            pl.BlockSpec((rb, n), lambda i: (i, 0)),
            pl.BlockSpec((1, n), lambda i: (0, 0)),
            pl.BlockSpec((1, n), lambda i: (0, 0)),
        ],
        out_specs=pl.BlockSpec((rb, n), lambda i: (i, 0)),
        out_shape=jax.ShapeDtypeStruct((b, n), jnp.float32),
    )(x_input, ln_gamma.reshape(1, n), ln_beta.reshape(1, n))


@jax.jit
def kernel(x_input, ln_gamma, ln_beta):
    sc_out = _sc_kernel(x_input[96:], ln_gamma, ln_beta)
    tc_out = _tc_call(x_input[:96], ln_gamma, ln_beta)
    return jnp.concatenate([tc_out, sc_out], axis=0)


# speedup vs baseline: 2.8670x; 1.0065x over previous
import functools

import jax
import jax.numpy as jnp
import numpy as np
from jax import lax
from jax.experimental import pallas as pl
from jax.experimental.pallas import tpu as pltpu
from jax.experimental.pallas import tpu_sc as plsc

_SPARSITY = 0.15
_GAIN = 3.0
_EPS = 1e-5

_B = 32
_N = 32768
_K = max(int(_N * _SPARSITY), 2)
_NVREG = _N // 16
_MIN32 = np.int32(-2147483648)
_M7F = np.int32(0x7FFFFFFF)

_GDN = lax.GatherDimensionNumbers(
    offset_dims=(), collapsed_slice_dims=(0,), start_index_map=(0,))


def _perm(v, idx):
    return lax.gather(v, idx[:, None], dimension_numbers=_GDN,
                      slice_sizes=(1,),
                      mode=lax.GatherScatterMode.PROMISE_IN_BOUNDS)


def _butterfly(v, lanes, op):
    for sh in (8, 4, 2, 1):
        v = op(v, _perm(v, lanes ^ sh))
    return v


def _suffix16(v, lanes, zero):
    for sh in (1, 2, 4, 8):
        shifted = _perm(v, jnp.minimum(lanes + sh, 15))
        v = v + jnp.where(lanes + sh > 15, zero, shifted)
    return v


def _sortable(bits):
    return jnp.where(bits >= 0, bits, bits ^ _M7F) ^ _MIN32


def _sc_body(x_hbm, g_hbm, b_hbm, o_hbm, sbuf, gbuf, bbuf, hist):
    wid = lax.axis_index("s") * 2 + lax.axis_index("c")

    pltpu.sync_copy(g_hbm, gbuf)
    pltpu.sync_copy(b_hbm, bbuf)
    lanes = lax.iota(jnp.int32, 16)
    izero = jnp.zeros((16,), jnp.int32)
    ones = jnp.ones((16,), jnp.int32)

    def do_row(row, _):
        pltpu.sync_copy(x_hbm.at[row], sbuf)

        def stats(j, carry):
            accs = list(carry)
            for u in range(16):
                v = sbuf[pl.ds(j * 256 + u * 16, 16)]
                accs[u] = accs[u] + v
                accs[16 + u] = accs[16 + u] + v * v
            return tuple(accs)

        z16 = jnp.zeros((16,), jnp.float32)
        accs = lax.fori_loop(0, _NVREG // 16, stats, (z16,) * 32)
        acc = accs[0]
        acc2 = accs[16]
        for u in range(1, 16):
            acc = acc + accs[u]
            acc2 = acc2 + accs[16 + u]
        mean = _butterfly(acc, lanes, jnp.add) * (1.0 / _N)
        ssq = _butterfly(acc2, lanes, jnp.add)
        var = jnp.maximum(ssq * (1.0 / _N) - mean * mean, 0.0) + _EPS
        vb = lax.bitcast_convert_type(var, jnp.int32)
        y = lax.bitcast_convert_type(np.int32(0x5F3759DF) - (vb >> 1),
                                     jnp.float32)
        for _i in range(4):
            y = y * (1.5 - 0.5 * var * y * y)
        istd = y

        def zero_hist(j, _c):
            for u in range(8):
                hist[pl.ds(j * 128 + u * 16, 16)] = izero
            return 0

        lanes16 = lanes * 16
        prefix = izero
        rank = jnp.full((16,), _K, jnp.int32)
        m1 = jnp.full((16,), _MIN32, jnp.int32)
        m2 = jnp.full((16,), _MIN32, jnp.int32)

        for level in range(4):
            sh = 24 - 8 * level
            lax.fori_loop(0, 34, zero_hist, 0)

            if level == 0:
                def hpass0(j, carry):
                    cm1, cm2 = carry
                    for u in range(16):
                        sl = pl.ds(j * 256 + u * 16, 16)
                        xn = (sbuf[sl] - mean) * istd * gbuf[sl] + bbuf[sl]
                        bits = lax.bitcast_convert_type(xn, jnp.int32)
                        s = _sortable(bits)
                        sbuf[sl] = lax.bitcast_convert_type(s, jnp.float32)
                        b = lax.shift_right_logical(s, 24)
                        plsc.addupdate_scatter(hist, [(b << 4) | lanes], ones)
                        sb = s ^ _MIN32
                        cm2 = jnp.maximum(cm2, jnp.minimum(cm1, sb))
                        cm1 = jnp.maximum(cm1, sb)
                    return cm1, cm2

                m1, m2 = lax.fori_loop(0, _NVREG // 16, hpass0, (m1, m2))
            else:
                pref = prefix

                def hpass(j, _c):
                    for u in range(16):
                        s = lax.bitcast_convert_type(
                            sbuf[pl.ds(j * 256 + u * 16, 16)], jnp.int32)
                        b = lax.shift_right_logical(s, sh) & 255
                        keep = lax.shift_right_logical(s, sh + 8) == pref
                        plsc.addupdate_scatter(hist, [(b << 4) | lanes],
                                               ones, mask=keep)
                    return 0

                lax.fori_loop(0, _NVREG // 16, hpass, 0)

            for g in range(16):
                cv = hist[pl.ds(g * 256, 16)]
                for j in range(1, 16):
                    cv = cv + hist[pl.ds(g * 256 + j * 16, 16)]
                hist[pl.ds(4096 + g * 16, 16)] = cv
            G = izero
            for l in range(16):
                G = G + plsc.load_gather(hist, [4096 + lanes16 + l])
            S = _suffix16(G, lanes, izero)
            hitg = S >= rank
            grp = plsc.all_reduce_population_count(hitg) - 1
            above_g = _butterfly(jnp.where(hitg, 0, S), lanes, jnp.maximum)

            bt = izero
            base = grp * 256 + lanes16
            for l in range(16):
                bt = bt + plsc.load_gather(hist, [base + l])
            rank2 = rank - above_g
            S2 = _suffix16(bt, lanes, izero)
            hitb = S2 >= rank2
            bloc = plsc.all_reduce_population_count(hitb) - 1
            above_b = _butterfly(jnp.where(hitb, 0, S2), lanes, jnp.maximum)
            prefix = (prefix << 8) | ((grp << 4) | bloc)
            rank = rank2 - above_b

        thr_b = prefix ^ _MIN32

        m1s = _butterfly(m1, lanes, jnp.maximum)
        c1 = plsc.all_reduce_population_count(m1 == m1s)
        mbig = jnp.full((16,), np.int32(-2147483647), jnp.int32)
        strict2 = _butterfly(jnp.where(m1 == m1s, mbig, m1), lanes,
                             jnp.maximum)
        m2s = _butterfly(m2, lanes, jnp.maximum)
        second = jnp.where(c1 >= 2, m1s, jnp.maximum(strict2, m2s))
        u1 = jnp.where(m1s >= 0, m1s, m1s ^ _M7F)
        u2 = jnp.where(second >= 0, second, second ^ _M7F)
        f1 = lax.bitcast_convert_type(u1, jnp.float32)
        f2 = lax.bitcast_convert_type(u2, jnp.float32)
        gain = _GAIN / (1.0 + jnp.exp(f2 - f1)) + 1.0

        def opass(j, _c):
            for u in range(16):
                sl = pl.ds(j * 256 + u * 16, 16)
                s = lax.bitcast_convert_type(sbuf[sl], jnp.int32)
                sb = s ^ _MIN32
                keep = sb >= thr_b
                w = jnp.where(sb >= 0, sb, sb ^ _M7F)
                xn = lax.bitcast_convert_type(w, jnp.float32)
                sbuf[sl] = jnp.where(keep, xn * gain, 0.0)
            return 0

        lax.fori_loop(0, _NVREG // 16, opass, 0)
        pltpu.sync_copy(sbuf, o_hbm.at[row])
        return 0

    do_row(wid, 0)


def _sc_kernel(x_input, ln_gamma, ln_beta):
    mesh = plsc.VectorSubcoreMesh(core_axis_name="c", subcore_axis_name="s")
    fn = pl.kernel(
        _sc_body,
        out_type=jax.ShapeDtypeStruct((_B, _N), jnp.float32),
        mesh=mesh,
        compiler_params=pltpu.CompilerParams(needs_layout_passes=False),
        scratch_types=[
            pltpu.VMEM((_N,), jnp.float32),
            pltpu.VMEM((_N,), jnp.float32),
            pltpu.VMEM((_N,), jnp.float32),
            pltpu.VMEM((4352,), jnp.int32),
        ],
    )
    return fn(x_input, ln_gamma, ln_beta)


def _tc_kernel(x_ref, g_ref, b_ref, o_ref, *, k):
    x = x_ref[...]
    rows = x.shape[0]

    mean = jnp.mean(x, axis=1, keepdims=True)
    xc = x - mean
    var = jnp.mean(xc * xc, axis=1, keepdims=True)
    xn = xc * jax.lax.rsqrt(var + _EPS)
    xn = xn * g_ref[...] + b_ref[...]

    u = jax.lax.bitcast_convert_type(xn, jnp.uint32)
    neg = u >= jnp.uint32(0x80000000)
    s = jnp.where(neg, ~u, u | jnp.uint32(0x80000000))

    lo0 = jnp.zeros((rows, 1), jnp.uint32)
    hi0 = jnp.full((rows, 1), 0xFFFFFFFF, jnp.uint32)

    def cond(carry):
        i, lo, hi = carry
        return jnp.logical_and(i < 32, jnp.logical_not(jnp.all(lo == hi)))

    def body(carry):
        i, lo, hi = carry
        d = hi - lo
        mid = lo + (d >> 1) + (d & jnp.uint32(1))
        cnt = jnp.sum((s >= mid).astype(jnp.int32), axis=1, keepdims=True)
        pred = cnt >= k
        lo = jnp.where(pred, mid, lo)
        hi = jnp.where(cnt == k, mid, jnp.where(pred, hi, mid - jnp.uint32(1)))
        return i + 1, lo, hi

    _, lo, _ = jax.lax.while_loop(cond, body, (jnp.int32(0), lo0, hi0))
    keep = s >= lo

    m1 = jnp.max(xn, axis=1, keepdims=True)
    is_max = xn == m1
    nmax = jnp.sum(is_max.astype(jnp.int32), axis=1, keepdims=True)
    m2_strict = jnp.max(jnp.where(is_max, -jnp.inf, xn), axis=1, keepdims=True)
    m2 = jnp.where(nmax >= 2, m1, m2_strict)
    gain = jax.nn.sigmoid(m1 - m2) * _GAIN + 1.0

    o_ref[...] = jnp.where(keep, xn * gain, 0.0)


def _tc_call(x_input, ln_gamma, ln_beta):
    b, n = x_input.shape
    k = max(int(n * _SPARSITY), 2)
    rb = 32
    grid = (b // rb,)
    body = functools.partial(_tc_kernel, k=k)
    return pl.pallas_call(
        body,
        grid=grid,
        in_specs=[
            pl.BlockSpec((rb, n), lambda i: (i, 0)),
            pl.BlockSpec((1, n), lambda i: (0, 0)),
            pl.BlockSpec((1, n), lambda i: (0, 0)),
        ],
        out_specs=pl.BlockSpec((rb, n), lambda i: (i, 0)),
        out_shape=jax.ShapeDtypeStruct((b, n), jnp.float32),
    )(x_input, ln_gamma.reshape(1, n), ln_beta.reshape(1, n))


@jax.jit
def kernel(x_input, ln_gamma, ln_beta):
    sc_out = _sc_kernel(x_input[96:], ln_gamma, ln_beta)
    tc_out = _tc_call(x_input[:96], ln_gamma, ln_beta)
    return jnp.concatenate([tc_out, sc_out], axis=0)
